# baseline (speedup 1.0000x reference)
import functools

import jax
import jax.numpy as jnp
from jax import lax
from jax.experimental import pallas as pl
from jax.experimental.pallas import tpu as pltpu
from jax.experimental.pallas import tpu_sc as plsc

D = 64
K = 5
PHASES = K + 1
SECS = PHASES + 1
NC = 2
NS = 16
NW = NC * NS
CHUNK = 64
LANES = 16
BLKV = 32768
HALF = BLKV // 2
QUAR = BLKV // 4
SHB = BLKV.bit_length() - 1
SHQ = SHB - 2


@functools.lru_cache(maxsize=None)
def _make_transpose_kernel(V: int):
    grid = (V + BLKV - 1) // BLKV

    def body(x_ref, o_ref):
        x = x_ref[...]
        blo = lax.bitcast_convert_type(x[0:D // 2, :], jnp.uint32)
        bhi = lax.bitcast_convert_type(x[D // 2:, :], jnp.uint32)
        packed = lax.bitcast_convert_type(
            (bhi & jnp.uint32(0xFFFF0000)) | (blo >> 16), jnp.float32)
        t = packed.T
        o_ref[...] = jnp.concatenate(
            [t[q * QUAR:(q + 1) * QUAR] for q in range(4)], axis=1)

    return pl.pallas_call(
        body,
        grid=(grid,),
        in_specs=[pl.BlockSpec((D, BLKV), lambda j: (0, j))],
        out_specs=pl.BlockSpec((QUAR, 2 * D), lambda j: (j, 0)),
        out_shape=jax.ShapeDtypeStruct((grid * QUAR, 2 * D), jnp.float32),
    )


@functools.lru_cache(maxsize=None)
def _make_scores_kernel(B: int):
    BW = B // NW
    NCH = BW // CHUNK
    mesh = plsc.VectorSubcoreMesh(core_axis_name="c", subcore_axis_name="s")

    @functools.partial(
        pl.kernel,
        out_type=jax.ShapeDtypeStruct((8, B), jnp.float32),
        mesh=mesh,
        compiler_params=pltpu.CompilerParams(needs_layout_passes=False),
        scratch_types=[
            pltpu.VMEM((SECS * BW,), jnp.int32),
            pltpu.VMEM((SECS * BW,), jnp.int32),
            pltpu.VMEM((2, SECS * CHUNK, 2 * D), jnp.float32),
            pltpu.VMEM((8, BW), jnp.float32),
            pltpu.SemaphoreType.DMA,
            pltpu.SemaphoreType.DMA,
        ],
    )
    def scores_kernel(userL, bookL, uid_hbm, pid_hbm, nid_hbm, out_hbm,
                      ids_v, phys_v, rows_v, scores_v, sem0, sem1):
        wid = lax.axis_index("s") * NC + lax.axis_index("c")
        base = wid * BW

        pltpu.sync_copy(uid_hbm.at[pl.ds(base, BW)], ids_v.at[pl.ds(0, BW)])
        pltpu.sync_copy(pid_hbm.at[pl.ds(base, BW)], ids_v.at[pl.ds(BW, BW)])
        for kk in range(K):
            pltpu.sync_copy(nid_hbm.at[pl.ds(kk * B + base, BW)],
                            ids_v.at[pl.ds((2 + kk) * BW, BW)])

        GPC = SECS * CHUNK // LANES
        GPS = CHUNK // LANES

        @plsc.parallel_loop(0, SECS * BW // LANES, 1, unroll=4)
        def _(j):
            c = j // GPC
            rem = j - c * GPC
            s = rem // GPS
            g = rem - s * GPS
            v = ids_v[pl.ds(s * BW + c * CHUNK + g * LANES, LANES)]
            phys_v[pl.ds(j * LANES, LANES)] = (
                ((v >> SHB) << SHQ) | (v & (QUAR - 1)))

        sems = (sem0, sem1)

        def issue(c):
            buf = c % 2
            bi = c * SECS * CHUNK
            sem = sems[buf]
            cps = [pltpu.async_copy(
                userL.at[phys_v.at[pl.ds(bi, CHUNK)]],
                rows_v.at[buf, pl.ds(0, CHUNK)], sem)]
            for q in range(3):
                off = CHUNK + q * 2 * CHUNK
                cps.append(pltpu.async_copy(
                    bookL.at[phys_v.at[pl.ds(bi + off, 2 * CHUNK)]],
                    rows_v.at[buf, pl.ds(off, 2 * CHUNK)], sem))
            return cps

        lane = lax.iota(jnp.int32, LANES)
        zeros = jnp.zeros((LANES,), jnp.float32)
        MASKHI = jnp.int32(-65536)

        def compute(c):
            buf = c % 2
            cb = c * CHUNK

            @plsc.parallel_loop(0, CHUNK // LANES, 1)
            def _(g):
                row_iota = g * LANES + lane
                pars = []
                rows = []
                for sec in range(SECS):
                    idv = ids_v[pl.ds(sec * BW + cb + g * LANES, LANES)]
                    pars.append(((idv >> SHQ) & 3) << 5)
                    rows.append(sec * CHUNK + row_iota)

                def dbody(t, accs):
                    dv = (lane + t) & (D // 2 - 1)
                    uw = plsc.load_gather(rows_v.at[buf],
                                          [rows[0], pars[0] + dv])
                    ub = plsc.bitcast(uw, jnp.int32)
                    ulo = plsc.bitcast(ub << 16, jnp.float32)
                    uhi = plsc.bitcast(ub & MASKHI, jnp.float32)
                    new = []
                    for p in range(PHASES):
                        vw = plsc.load_gather(rows_v.at[buf],
                                              [rows[1 + p], pars[1 + p] + dv])
                        vb = plsc.bitcast(vw, jnp.int32)
                        vlo = plsc.bitcast(vb << 16, jnp.float32)
                        vhi = plsc.bitcast(vb & MASKHI, jnp.float32)
                        new.append(accs[p] + (ulo * vlo + uhi * vhi))
                    return tuple(new)

                accs = plsc.parallel_loop(
                    0, D // 2, 1, unroll=4,
                    carry=tuple(zeros for _ in range(PHASES)))(dbody)
                for p in range(PHASES):
                    scores_v[p, pl.ds(cb + g * LANES, LANES)] = accs[p]

        pending = {0: issue(0)}
        for c in range(NCH):
            if c + 1 < NCH:
                pending[c + 1] = issue(c + 1)
            for cp in pending.pop(c):
                cp.wait()
            compute(c)

        pltpu.sync_copy(scores_v, out_hbm.at[:, pl.ds(base, BW)])

    return scores_kernel


@functools.lru_cache(maxsize=None)
def _make_loss_kernel(B: int):
    def loss_body(scores_ref, out_ref):
        x = scores_ref[...]
        pos = x[0:1, :]
        neg = x[1:PHASES, :]
        lp = jnp.log(jax.nn.sigmoid(pos) + 1e-10)
        ln = jnp.log(jax.nn.sigmoid(-neg) + 1e-10)
        out_ref[0, 0] = -(jnp.sum(lp) + jnp.sum(ln)) / B

    return pl.pallas_call(
        loss_body,
        out_shape=jax.ShapeDtypeStruct((1, 1), jnp.float32),
        out_specs=pl.BlockSpec(memory_space=pltpu.SMEM),
    )


def kernel(user_embed, book_embed, user_ids, pos_book_ids, neg_book_ids):
    B = user_ids.shape[0]
    NU, V = user_embed.shape[0], book_embed.shape[0]
    uid = user_ids.astype(jnp.int32)
    pid = pos_book_ids.astype(jnp.int32)
    nidT = neg_book_ids.astype(jnp.int32).T.reshape(-1)
    userL = _make_transpose_kernel(NU)(user_embed.T)
    bookL = _make_transpose_kernel(V)(book_embed.T)
    scores = _make_scores_kernel(B)(userL, bookL, uid, pid, nidT)
    loss = _make_loss_kernel(B)(scores)
    return loss[0, 0]


# speedup vs baseline: 1.1231x; 1.1231x over previous
import functools

import jax
import jax.numpy as jnp
from jax import lax
from jax.experimental import pallas as pl
from jax.experimental.pallas import tpu as pltpu
from jax.experimental.pallas import tpu_sc as plsc

D = 64
K = 5
PHASES = K + 1
SECS = PHASES + 1
NC = 2
NS = 16
NW = NC * NS
CHUNK = 64
LANES = 16
BLKV = 32768
HALF = BLKV // 2
SHB = BLKV.bit_length() - 1


@functools.lru_cache(maxsize=None)
def _make_transpose_kernel(V: int):
    grid = (V + BLKV - 1) // BLKV

    def body(x_ref, o_ref):
        t = x_ref[...].T
        o_ref[:, 0:D] = t[0:HALF, :]
        o_ref[:, D:2 * D] = t[HALF:, :]

    return pl.pallas_call(
        body,
        grid=(grid,),
        in_specs=[pl.BlockSpec((D, BLKV), lambda j: (0, j))],
        out_specs=pl.BlockSpec((HALF, 2 * D), lambda j: (j, 0)),
        out_shape=jax.ShapeDtypeStruct((grid * HALF, 2 * D), jnp.float32),
    )


@functools.lru_cache(maxsize=None)
def _make_scores_kernel(B: int):
    BW = B // NW
    NCH = BW // CHUNK
    mesh = plsc.VectorSubcoreMesh(core_axis_name="c", subcore_axis_name="s")

    @functools.partial(
        pl.kernel,
        out_type=jax.ShapeDtypeStruct((8, B), jnp.float32),
        mesh=mesh,
        compiler_params=pltpu.CompilerParams(needs_layout_passes=False),
        scratch_types=[
            pltpu.VMEM((SECS * BW,), jnp.int32),
            pltpu.VMEM((SECS * BW,), jnp.int32),
            pltpu.VMEM((2, SECS * CHUNK, 2 * D), jnp.float32),
            pltpu.VMEM((8, BW), jnp.float32),
            pltpu.SemaphoreType.DMA,
            pltpu.SemaphoreType.DMA,
        ],
    )
    def scores_kernel(userL, bookL, uid_hbm, pid_hbm, nid_hbm, out_hbm,
                      ids_v, phys_v, rows_v, scores_v, sem0, sem1):
        wid = lax.axis_index("s") * NC + lax.axis_index("c")
        base = wid * BW

        pltpu.sync_copy(uid_hbm.at[pl.ds(base, BW)], ids_v.at[pl.ds(0, BW)])
        pltpu.sync_copy(pid_hbm.at[pl.ds(base, BW)], ids_v.at[pl.ds(BW, BW)])
        for kk in range(K):
            pltpu.sync_copy(nid_hbm.at[pl.ds(kk * B + base, BW)],
                            ids_v.at[pl.ds((2 + kk) * BW, BW)])

        GPC = SECS * CHUNK // LANES
        GPS = CHUNK // LANES

        @plsc.parallel_loop(0, SECS * BW // LANES, 1, unroll=4)
        def _(j):
            c = j // GPC
            rem = j - c * GPC
            s = rem // GPS
            g = rem - s * GPS
            v = ids_v[pl.ds(s * BW + c * CHUNK + g * LANES, LANES)]
            phys_v[pl.ds(j * LANES, LANES)] = (
                ((v >> SHB) << (SHB - 1)) | (v & (HALF - 1)))

        sems = (sem0, sem1)

        def issue(c):
            buf = c % 2
            bi = c * SECS * CHUNK
            sem = sems[buf]
            cps = [pltpu.async_copy(
                userL.at[phys_v.at[pl.ds(bi, CHUNK)]],
                rows_v.at[buf, pl.ds(0, CHUNK)], sem)]
            for q in range(3):
                off = CHUNK + q * 2 * CHUNK
                cps.append(pltpu.async_copy(
                    bookL.at[phys_v.at[pl.ds(bi + off, 2 * CHUNK)]],
                    rows_v.at[buf, pl.ds(off, 2 * CHUNK)], sem))
            return cps

        lane = lax.iota(jnp.int32, LANES)
        zeros = jnp.zeros((LANES,), jnp.float32)

        def compute(c):
            buf = c % 2
            cb = c * CHUNK

            @plsc.parallel_loop(0, CHUNK // LANES, 1)
            def _(g):
                row_iota = g * LANES + lane
                pars = []
                rows = []
                for sec in range(SECS):
                    idv = ids_v[pl.ds(sec * BW + cb + g * LANES, LANES)]
                    pars.append(((idv >> (SHB - 1)) & 1) << 6)
                    rows.append(sec * CHUNK + row_iota)

                def dbody(t, accs):
                    dv = (lane + t) & (D - 1)
                    u_d = plsc.load_gather(rows_v.at[buf],
                                           [rows[0], pars[0] + dv])
                    new = []
                    for p in range(PHASES):
                        v_d = plsc.load_gather(rows_v.at[buf],
                                               [rows[1 + p], pars[1 + p] + dv])
                        new.append(accs[p] + u_d * v_d)
                    return tuple(new)

                accs = plsc.parallel_loop(
                    0, D, 1, unroll=4,
                    carry=tuple(zeros for _ in range(PHASES)))(dbody)
                for p in range(PHASES):
                    scores_v[p, pl.ds(cb + g * LANES, LANES)] = accs[p]

        pending = {0: issue(0)}
        for c in range(NCH):
            if c + 1 < NCH:
                pending[c + 1] = issue(c + 1)
            for cp in pending.pop(c):
                cp.wait()
            compute(c)

        pltpu.sync_copy(scores_v, out_hbm.at[:, pl.ds(base, BW)])

    return scores_kernel


@functools.lru_cache(maxsize=None)
def _make_loss_kernel(B: int):
    def loss_body(scores_ref, out_ref):
        x = scores_ref[...]
        pos = x[0:1, :]
        neg = x[1:PHASES, :]
        lp = jnp.log(jax.nn.sigmoid(pos) + 1e-10)
        ln = jnp.log(jax.nn.sigmoid(-neg) + 1e-10)
        out_ref[0, 0] = -(jnp.sum(lp) + jnp.sum(ln)) / B

    return pl.pallas_call(
        loss_body,
        out_shape=jax.ShapeDtypeStruct((1, 1), jnp.float32),
        out_specs=pl.BlockSpec(memory_space=pltpu.SMEM),
    )


def kernel(user_embed, book_embed, user_ids, pos_book_ids, neg_book_ids):
    B = user_ids.shape[0]
    NU, V = user_embed.shape[0], book_embed.shape[0]
    uid = user_ids.astype(jnp.int32)
    pid = pos_book_ids.astype(jnp.int32)
    nidT = neg_book_ids.astype(jnp.int32).T.reshape(-1)
    userL = _make_transpose_kernel(NU)(user_embed.T)
    bookL = _make_transpose_kernel(V)(book_embed.T)
    scores = _make_scores_kernel(B)(userL, bookL, uid, pid, nidT)
    loss = _make_loss_kernel(B)(scores)
    return loss[0, 0]


# speedup vs baseline: 1.4165x; 1.2612x over previous
import functools

import jax
import jax.numpy as jnp
from jax import lax
from jax.experimental import pallas as pl
from jax.experimental.pallas import tpu as pltpu
from jax.experimental.pallas import tpu_sc as plsc

D = 64
K = 5
PHASES = K + 1
SECS = PHASES + 1
NC = 2
NS = 16
NW = NC * NS
CHUNK = 64
LANES = 16
BLKV = 32768
HALF = BLKV // 2
SHB = BLKV.bit_length() - 1


@functools.lru_cache(maxsize=None)
def _make_transpose_kernel(V: int):
    grid = (V + BLKV - 1) // BLKV

    def body(x_ref, o_ref):
        rr = lax.broadcasted_iota(jnp.int32, (2 * D, 2 * D), 0)
        cc = lax.broadcasted_iota(jnp.int32, (2 * D, 2 * D), 1)
        eye = (rr == cc).astype(jnp.float32)
        x = x_ref[...]
        y = jnp.concatenate([x[:, :HALF], x[:, HALF:]], axis=0)
        o_ref[...] = lax.dot_general(
            y, eye, (((0,), (0,)), ((), ())),
            preferred_element_type=jnp.float32)

    return pl.pallas_call(
        body,
        grid=(grid,),
        in_specs=[pl.BlockSpec((D, BLKV), lambda j: (0, j))],
        out_specs=pl.BlockSpec((HALF, 2 * D), lambda j: (j, 0)),
        out_shape=jax.ShapeDtypeStruct((grid * HALF, 2 * D), jnp.float32),
        compiler_params=pltpu.CompilerParams(
            fuse_transposed_lhs_in_matmul=True),
    )


@functools.lru_cache(maxsize=None)
def _make_scores_kernel(B: int):
    BW = B // NW
    NCH = BW // CHUNK
    mesh = plsc.VectorSubcoreMesh(core_axis_name="c", subcore_axis_name="s")

    @functools.partial(
        pl.kernel,
        out_type=jax.ShapeDtypeStruct((8, B), jnp.float32),
        mesh=mesh,
        compiler_params=pltpu.CompilerParams(needs_layout_passes=False),
        scratch_types=[
            pltpu.VMEM((SECS * BW,), jnp.int32),
            pltpu.VMEM((SECS * BW,), jnp.int32),
            pltpu.VMEM((2, SECS * CHUNK, 2 * D), jnp.float32),
            pltpu.VMEM((8, BW), jnp.float32),
            pltpu.SemaphoreType.DMA,
            pltpu.SemaphoreType.DMA,
        ],
    )
    def scores_kernel(userL, bookL, uid_hbm, pid_hbm, nid_hbm, out_hbm,
                      ids_v, phys_v, rows_v, scores_v, sem0, sem1):
        wid = lax.axis_index("s") * NC + lax.axis_index("c")
        base = wid * BW

        pltpu.sync_copy(uid_hbm.at[pl.ds(base, BW)], ids_v.at[pl.ds(0, BW)])
        pltpu.sync_copy(pid_hbm.at[pl.ds(base, BW)], ids_v.at[pl.ds(BW, BW)])
        for kk in range(K):
            pltpu.sync_copy(nid_hbm.at[pl.ds(kk * B + base, BW)],
                            ids_v.at[pl.ds((2 + kk) * BW, BW)])

        GPC = SECS * CHUNK // LANES
        GPS = CHUNK // LANES

        @plsc.parallel_loop(0, SECS * BW // LANES, 1, unroll=4)
        def _(j):
            c = j // GPC
            rem = j - c * GPC
            s = rem // GPS
            g = rem - s * GPS
            v = ids_v[pl.ds(s * BW + c * CHUNK + g * LANES, LANES)]
            phys_v[pl.ds(j * LANES, LANES)] = (
                ((v >> SHB) << (SHB - 1)) | (v & (HALF - 1)))

        sems = (sem0, sem1)

        def issue(c):
            buf = c % 2
            bi = c * SECS * CHUNK
            sem = sems[buf]
            cps = [pltpu.async_copy(
                userL.at[phys_v.at[pl.ds(bi, CHUNK)]],
                rows_v.at[buf, pl.ds(0, CHUNK)], sem)]
            for q in range(3):
                off = CHUNK + q * 2 * CHUNK
                cps.append(pltpu.async_copy(
                    bookL.at[phys_v.at[pl.ds(bi + off, 2 * CHUNK)]],
                    rows_v.at[buf, pl.ds(off, 2 * CHUNK)], sem))
            return cps

        lane = lax.iota(jnp.int32, LANES)
        zeros = jnp.zeros((LANES,), jnp.float32)

        def compute(c):
            buf = c % 2
            cb = c * CHUNK

            @plsc.parallel_loop(0, CHUNK // LANES, 1)
            def _(g):
                row_iota = g * LANES + lane
                pars = []
                rows = []
                for sec in range(SECS):
                    idv = ids_v[pl.ds(sec * BW + cb + g * LANES, LANES)]
                    pars.append(((idv >> (SHB - 1)) & 1) << 6)
                    rows.append(sec * CHUNK + row_iota)

                def dbody(t, accs):
                    dv = (lane + t) & (D - 1)
                    u_d = plsc.load_gather(rows_v.at[buf],
                                           [rows[0], pars[0] + dv])
                    new = []
                    for p in range(PHASES):
                        v_d = plsc.load_gather(rows_v.at[buf],
                                               [rows[1 + p], pars[1 + p] + dv])
                        new.append(accs[p] + u_d * v_d)
                    return tuple(new)

                accs = plsc.parallel_loop(
                    0, D, 1, unroll=4,
                    carry=tuple(zeros for _ in range(PHASES)))(dbody)
                for p in range(PHASES):
                    scores_v[p, pl.ds(cb + g * LANES, LANES)] = accs[p]

        pending = {0: issue(0)}
        for c in range(NCH):
            if c + 1 < NCH:
                pending[c + 1] = issue(c + 1)
            for cp in pending.pop(c):
                cp.wait()
            compute(c)

        pltpu.sync_copy(scores_v, out_hbm.at[:, pl.ds(base, BW)])

    return scores_kernel


@functools.lru_cache(maxsize=None)
def _make_loss_kernel(B: int):
    def loss_body(scores_ref, out_ref):
        x = scores_ref[...]
        pos = x[0:1, :]
        neg = x[1:PHASES, :]
        lp = jnp.log(jax.nn.sigmoid(pos) + 1e-10)
        ln = jnp.log(jax.nn.sigmoid(-neg) + 1e-10)
        out_ref[0, 0] = -(jnp.sum(lp) + jnp.sum(ln)) / B

    return pl.pallas_call(
        loss_body,
        out_shape=jax.ShapeDtypeStruct((1, 1), jnp.float32),
        out_specs=pl.BlockSpec(memory_space=pltpu.SMEM),
    )


def kernel(user_embed, book_embed, user_ids, pos_book_ids, neg_book_ids):
    B = user_ids.shape[0]
    NU, V = user_embed.shape[0], book_embed.shape[0]
    uid = user_ids.astype(jnp.int32)
    pid = pos_book_ids.astype(jnp.int32)
    nidT = neg_book_ids.astype(jnp.int32).T.reshape(-1)
    userL = _make_transpose_kernel(NU)(user_embed.T)
    bookL = _make_transpose_kernel(V)(book_embed.T)
    scores = _make_scores_kernel(B)(userL, bookL, uid, pid, nidT)
    loss = _make_loss_kernel(B)(scores)
    return loss[0, 0]
